# baseline (speedup 1.0000x reference)
import functools

import jax
import jax.numpy as jnp
from jax import lax
from jax.experimental import pallas as pl
from jax.experimental.pallas import tpu as pltpu
from jax.experimental.pallas import tpu_sc as plsc

_N = 10000
_E = 320000
_H = 128
_NC = 2
_NS = 16
_NW = _NC * _NS
_BX = 64
_BE = 1600
_BN = 2000


def _node_prep_body(x_ref, gn_ref, bn_ref, wes_ref, wed_ref, wn1s_ref,
                    xn_ref, p1_ref, p2_ref, p3_ref):
    x = x_ref[...]
    mu = jnp.mean(x, axis=1, keepdims=True)
    xc = x - mu
    var = jnp.mean(xc * xc, axis=1, keepdims=True)
    xn = xc * lax.rsqrt(var + 1e-5) * gn_ref[...] + bn_ref[...]
    xn_ref[...] = xn
    p1_ref[...] = jnp.dot(xn, wes_ref[...], preferred_element_type=jnp.float32)
    p2_ref[...] = jnp.dot(xn, wed_ref[...], preferred_element_type=jnp.float32)
    p3_ref[...] = jnp.dot(xn, wn1s_ref[...], preferred_element_type=jnp.float32)


def _edge_mlp_body(ea_ref, gs_ref, ge_ref, bee_ref, be_ref, bn1_ref,
                   bem_ref, wee_ref, wn1e_ref, wem_ref, eo_ref, c_ref):
    gs = gs_ref[...]
    ea = ea_ref[0]
    mu = jnp.mean(ea, axis=1, keepdims=True)
    xc = ea - mu
    var = jnp.mean(xc * xc, axis=1, keepdims=True)
    en = xc * lax.rsqrt(var + 1e-5) * ge_ref[...] + bee_ref[...]
    e = jnp.maximum(
        gs + jnp.dot(en, wee_ref[...], preferred_element_type=jnp.float32)
        + be_ref[...], 0.0)
    c_ref[...] = jnp.dot(e, wn1e_ref[...], preferred_element_type=jnp.float32) + bn1_ref[...]
    se = e / (1.0 + jnp.exp(-e))
    eo_ref[...] = ea + jnp.dot(se, wem_ref[...], preferred_element_type=jnp.float32) + bem_ref[...]


def _node_update_body(sp_ref, cp_ref, xn_ref, x_ref, wx_ref, wa_ref, wm_ref,
                      bn2_ref, bnm_ref, xo_ref):
    sums = jnp.sum(sp_ref[...], axis=0)
    cnt = jnp.sum(cp_ref[...], axis=0)[:, 0:1]
    agg = sums / jnp.maximum(cnt, 1.0)
    h = jnp.maximum(
        jnp.dot(xn_ref[...], wx_ref[...], preferred_element_type=jnp.float32)
        + jnp.dot(agg, wa_ref[...], preferred_element_type=jnp.float32)
        + bn2_ref[...], 0.0)
    sh = h / (1.0 + jnp.exp(-h))
    xo_ref[...] = x_ref[...] + jnp.dot(sh, wm_ref[...], preferred_element_type=jnp.float32) + bnm_ref[...]


def _tile_chunks(n, bx):
    assert n % _NS == 0
    rpt = n // _NS
    rptc = (rpt + 7) // 8 * 8
    assert rptc <= rpt + ((_NS - 1) * rpt) % 8
    chunks = []
    left = rptc
    while left > 0:
        chunks.append(min(bx, left))
        left -= chunks[-1]
    return rpt, chunks


def _build_gather(n, e, h, bx, interpret=False):
    nblk = e // bx
    rpt, chunks = _tile_chunks(n, bx)
    mesh = plsc.VectorSubcoreMesh(core_axis_name="c", subcore_axis_name="s")

    @functools.partial(
        pl.kernel,
        out_type=(jax.ShapeDtypeStruct((e, h), jnp.float32),
                  jax.ShapeDtypeStruct((_NC * n, h), jnp.float32)),
        mesh=mesh,
        scratch_types=[
            pltpu.VMEM((bx,), jnp.int32),
            pltpu.VMEM((bx,), jnp.int32),
            pltpu.VMEM((bx,), jnp.int32),
            pltpu.VMEM((bx,), jnp.int32),
            pltpu.VMEM((bx, h), jnp.float32),
            pltpu.VMEM((bx, h), jnp.float32),
            pltpu.VMEM((bx, h), jnp.float32),
            pltpu.VMEM((bx, h), jnp.float32),
            pltpu.VMEM((bx, h), jnp.float32),
            pltpu.VMEM_SHARED((n, h), jnp.float32),
            pltpu.SemaphoreType.DMA,
            pltpu.SemaphoreType.DMA,
            pltpu.SemaphoreType.DMA,
            pltpu.SemaphoreType.DMA,
            pltpu.SemaphoreType.DMA,
            pltpu.SemaphoreType.DMA,
        ],
        interpret=interpret,
    )
    def gather_add(p1_hbm, p2_hbm, row_hbm, col_hbm, out_hbm, cnt_hbm,
                   idxr0, idxr1, idxc0, idxc1, bufa0, bufa1, bufb0, bufb1,
                   onesb, cntacc,
                   semi0, semi1, semg0, semg1, semw0, semw1):
        idxr = (idxr0, idxr1)
        idxc = (idxc0, idxc1)
        bufa = (bufa0, bufa1)
        bufb = (bufb0, bufb1)
        semi = (semi0, semi1)
        semg = (semg0, semg1)
        semw = (semw0, semw1)
        cid = lax.axis_index("c")
        sid = lax.axis_index("s")
        wid = sid * _NC + cid

        def zrow(i, c2):
            for jj in range(h // 16):
                onesb[i, pl.ds(jj * 16, 16)] = jnp.zeros((16,), jnp.float32)
            return c2

        lax.fori_loop(0, bx, zrow, 0)
        r0 = sid * rpt - lax.rem(sid * rpt, 8)
        ofs = 0
        for step in chunks:
            pltpu.sync_copy(onesb.at[pl.ds(0, step)], cntacc.at[pl.ds(r0 + ofs, step)])
            ofs += step

        def onerow(i, c2):
            for jj in range(h // 16):
                onesb[i, pl.ds(jj * 16, 16)] = jnp.full((16,), 1.0, jnp.float32)
            return c2

        lax.fori_loop(0, bx, onerow, 0)
        plsc.subcore_barrier()

        nblk_w = (nblk - 1 - wid) // _NW + 1

        def off_of(k):
            return pl.multiple_of((wid + k * _NW) * bx, bx)

        def issue_idx(k, s):
            pltpu.async_copy(row_hbm.at[pl.ds(off_of(k), bx)], idxr[s], semi[s])
            pltpu.async_copy(col_hbm.at[pl.ds(off_of(k), bx)], idxc[s], semi[s])

        def wait_idx(s):
            pltpu.make_async_copy(row_hbm.at[pl.ds(0, bx)], idxr[s], semi[s]).wait()
            pltpu.make_async_copy(col_hbm.at[pl.ds(0, bx)], idxc[s], semi[s]).wait()

        def issue_gather(s):
            pltpu.async_copy(p1_hbm.at[idxr[s]], bufa[s], semg[s])
            pltpu.async_copy(p2_hbm.at[idxc[s]], bufb[s], semg[s])

        def wait_gather(s):
            pltpu.make_async_copy(p1_hbm.at[idxr[s]], bufa[s], semg[s]).wait()
            pltpu.make_async_copy(p2_hbm.at[idxc[s]], bufb[s], semg[s]).wait()

        def issue_write(k, s):
            pltpu.async_copy(bufa[s], out_hbm.at[pl.ds(off_of(k), bx)], semw[s])

        def wait_write(s):
            pltpu.make_async_copy(bufa[s], out_hbm.at[pl.ds(0, bx)], semw[s]).wait()

        @pl.when(nblk_w > 0)
        def _():
            issue_idx(0, 0)

        @pl.when(nblk_w > 1)
        def _():
            issue_idx(1, 1)

        @pl.when(nblk_w > 0)
        def _():
            wait_idx(0)
            issue_gather(0)

        def outer(j, carry):
            for b in range(2):
                k = 2 * j + b
                p = b
                q = 1 - b
                kn = k + 1

                @pl.when(kn < nblk_w)
                def _():
                    wait_idx(q)

                @pl.when((k >= 1) & (k < nblk_w))
                def _():
                    wait_write(q)

                @pl.when(kn < nblk_w)
                def _():
                    issue_gather(q)

                @pl.when(k < nblk_w)
                def _():
                    wait_gather(p)

                    def addrow(i, c2):
                        for jj in range(h // 16):
                            sl = pl.ds(jj * 16, 16)
                            bufa[p][i, sl] = bufa[p][i, sl] + bufb[p][i, sl]
                        return c2

                    lax.fori_loop(0, bx, addrow, 0)
                    pltpu.sync_copy(onesb, cntacc.at[idxc[p]], add=True)

                @pl.when(k + 2 < nblk_w)
                def _():
                    issue_idx(k + 2, p)

                @pl.when(k < nblk_w)
                def _():
                    issue_write(k, p)
            return carry

        lax.fori_loop(0, (nblk_w + 1) // 2, outer, 0)

        last = nblk_w - 1

        @pl.when((last >= 0) & (lax.rem(last, 2) == 0))
        def _():
            wait_write(0)

        @pl.when((last >= 0) & (lax.rem(last, 2) == 1))
        def _():
            wait_write(1)

        plsc.subcore_barrier()

        ofs = 0
        for step in chunks:
            woff = pl.multiple_of(cid * n + r0 + ofs, 8)
            pltpu.sync_copy(cntacc.at[pl.ds(r0 + ofs, step)], onesb.at[pl.ds(0, step)])
            pltpu.sync_copy(onesb.at[pl.ds(0, step)], cnt_hbm.at[pl.ds(woff, step)])
            ofs += step

    return gather_add


def _build_scatter(n, e, h, bx, interpret=False):
    nblk = e // bx
    rpt, chunks = _tile_chunks(n, bx)
    mesh = plsc.VectorSubcoreMesh(core_axis_name="c", subcore_axis_name="s")

    @functools.partial(
        pl.kernel,
        out_type=jax.ShapeDtypeStruct((_NC * n, h), jnp.float32),
        mesh=mesh,
        scratch_types=[
            pltpu.VMEM((bx,), jnp.int32),
            pltpu.VMEM((bx,), jnp.int32),
            pltpu.VMEM((bx,), jnp.int32),
            pltpu.VMEM((bx,), jnp.int32),
            pltpu.VMEM((bx, h), jnp.float32),
            pltpu.VMEM((bx, h), jnp.float32),
            pltpu.VMEM((bx, h), jnp.float32),
            pltpu.VMEM((bx, h), jnp.float32),
            pltpu.VMEM_SHARED((n, h), jnp.float32),
            pltpu.SemaphoreType.DMA,
            pltpu.SemaphoreType.DMA,
            pltpu.SemaphoreType.DMA,
            pltpu.SemaphoreType.DMA,
            pltpu.SemaphoreType.DMA,
            pltpu.SemaphoreType.DMA,
        ],
        interpret=interpret,
    )
    def scatter_mean(p3_hbm, c_hbm, row_hbm, col_hbm, sums_hbm,
                     idxr0, idxr1, idxc0, idxc1, cbuf0, cbuf1, gbuf0, gbuf1,
                     acc, semi0, semi1, semc0, semc1, semg0, semg1):
        idxr = (idxr0, idxr1)
        idxc = (idxc0, idxc1)
        cbuf = (cbuf0, cbuf1)
        gbuf = (gbuf0, gbuf1)
        semi = (semi0, semi1)
        semc = (semc0, semc1)
        semg = (semg0, semg1)
        cid = lax.axis_index("c")
        sid = lax.axis_index("s")
        wid = sid * _NC + cid

        def zrow(i, c2):
            for j in range(h // 16):
                cbuf0[i, pl.ds(j * 16, 16)] = jnp.zeros((16,), jnp.float32)
            return c2

        lax.fori_loop(0, bx, zrow, 0)

        r0 = sid * rpt - lax.rem(sid * rpt, 8)
        ofs = 0
        for step in chunks:
            pltpu.sync_copy(cbuf0.at[pl.ds(0, step)], acc.at[pl.ds(r0 + ofs, step)])
            ofs += step
        plsc.subcore_barrier()

        nblk_w = (nblk - 1 - wid) // _NW + 1

        def off_of(k):
            return pl.multiple_of((wid + k * _NW) * bx, bx)

        def issue_idx(k, s):
            pltpu.async_copy(row_hbm.at[pl.ds(off_of(k), bx)], idxr[s], semi[s])
            pltpu.async_copy(col_hbm.at[pl.ds(off_of(k), bx)], idxc[s], semi[s])

        def wait_idx(s):
            pltpu.make_async_copy(row_hbm.at[pl.ds(0, bx)], idxr[s], semi[s]).wait()
            pltpu.make_async_copy(col_hbm.at[pl.ds(0, bx)], idxc[s], semi[s]).wait()

        def issue_data(k, s):
            pltpu.async_copy(c_hbm.at[pl.ds(off_of(k), bx)], cbuf[s], semc[s])
            pltpu.async_copy(p3_hbm.at[idxr[s]], gbuf[s], semg[s])

        def wait_data(s):
            pltpu.make_async_copy(c_hbm.at[pl.ds(0, bx)], cbuf[s], semc[s]).wait()
            pltpu.make_async_copy(p3_hbm.at[idxr[s]], gbuf[s], semg[s]).wait()

        @pl.when(nblk_w > 0)
        def _():
            issue_idx(0, 0)

        @pl.when(nblk_w > 1)
        def _():
            issue_idx(1, 1)

        @pl.when(nblk_w > 0)
        def _():
            wait_idx(0)
            issue_data(0, 0)

        def outer(j, carry):
            for b in range(2):
                k = 2 * j + b
                p = b
                q = 1 - b
                kn = k + 1

                @pl.when(kn < nblk_w)
                def _():
                    wait_idx(q)
                    issue_data(kn, q)

                @pl.when(k < nblk_w)
                def _():
                    wait_data(p)

                    def mrow(i, c2):
                        for jj in range(h // 16):
                            sl = pl.ds(jj * 16, 16)
                            cbuf[p][i, sl] = jnp.maximum(cbuf[p][i, sl] + gbuf[p][i, sl], 0.0)
                        return c2

                    lax.fori_loop(0, bx, mrow, 0)
                    pltpu.sync_copy(cbuf[p], acc.at[idxc[p]], add=True)

                @pl.when(k + 2 < nblk_w)
                def _():
                    issue_idx(k + 2, p)
            return carry

        lax.fori_loop(0, (nblk_w + 1) // 2, outer, 0)
        plsc.subcore_barrier()

        ofs = 0
        for step in chunks:
            woff = pl.multiple_of(cid * n + r0 + ofs, 8)
            pltpu.sync_copy(acc.at[pl.ds(r0 + ofs, step)], cbuf0.at[pl.ds(0, step)])
            pltpu.sync_copy(cbuf0.at[pl.ds(0, step)], sums_hbm.at[pl.ds(woff, step)])
            ofs += step

    return scatter_mean


_NSPLIT = 4


@functools.cache
def _gather_sc_cached():
    return _build_gather(_N, _E // _NSPLIT, _H, _BX)


@functools.cache
def _scatter_sc_cached():
    return _build_scatter(_N, _E // _NSPLIT, _H, 64)


def _gather_sc(p1, p2, row, col):
    return _gather_sc_cached()(p1, p2, row, col)


def _scatter_sc(p3, c, row, col):
    return _scatter_sc_cached()(p3, c, row, col)


def kernel(x, edge_index, edge_attr, u, batch, g_n, b_n, g_e, b_e, We, be,
           Wn1, bn1, Wn2, bn2, Wnm, bnm, Wem, bem):
    del u, batch
    row = edge_index[0]
    col = edge_index[1]
    wes, wed, wee = We[:_H], We[_H:2 * _H], We[2 * _H:]
    wn1s, wn1e = Wn1[:_H], Wn1[_H:]
    wn2x, wn2a = Wn2[:_H], Wn2[_H:]
    r2 = lambda v: v.reshape(1, _H)

    xn, p1, p2, p3 = pl.pallas_call(
        _node_prep_body,
        out_shape=[jax.ShapeDtypeStruct((_N, _H), jnp.float32)] * 4,
    )(x, r2(g_n), r2(b_n), wes, wed, wn1s)

    e2 = _E // _NSPLIT
    ea3 = edge_attr.reshape(_NSPLIT, e2, _H)
    wspec = pl.BlockSpec((_H, _H), lambda i: (0, 0))
    bspec = pl.BlockSpec((1, _H), lambda i: (0, 0))
    espec = pl.BlockSpec((_BE, _H), lambda i: (i, 0))

    gs_c = [_gather_sc(p1, p2, row[t * e2:(t + 1) * e2],
                       col[t * e2:(t + 1) * e2]) for t in range(_NSPLIT)]

    eo_c = []
    c_c = []
    for t in range(_NSPLIT):
        eo_t, c_t = pl.pallas_call(
            _edge_mlp_body,
            grid=(e2 // _BE,),
            in_specs=[pl.BlockSpec((1, _BE, _H), lambda i, t=t: (t, i, 0)),
                      espec, bspec, bspec, bspec, bspec, bspec,
                      wspec, wspec, wspec],
            out_specs=[espec, espec],
            out_shape=[jax.ShapeDtypeStruct((e2, _H), jnp.float32)] * 2,
        )(ea3, gs_c[t][0], r2(g_e), r2(b_e), r2(be), r2(bn1), r2(bem),
          wee, wn1e, Wem)
        eo_c.append(eo_t)
        c_c.append(c_t)
    eo = jnp.concatenate(eo_c, axis=0)

    sums_c = [_scatter_sc(p3, c_c[t], row[t * e2:(t + 1) * e2],
                          col[t * e2:(t + 1) * e2]) for t in range(_NSPLIT)]

    sp = jnp.concatenate(sums_c, axis=0).reshape(_NSPLIT * _NC, _N, _H)
    cp = jnp.concatenate([g[1] for g in gs_c], axis=0).reshape(
        _NSPLIT * _NC, _N, _H)

    nparts = _NSPLIT * _NC
    nspec = pl.BlockSpec((_BN, _H), lambda i: (i, 0))
    x_out = pl.pallas_call(
        _node_update_body,
        grid=(_N // _BN,),
        in_specs=[pl.BlockSpec((nparts, _BN, _H), lambda i: (0, i, 0)),
                  pl.BlockSpec((nparts, _BN, _H), lambda i: (0, i, 0)),
                  nspec, nspec, wspec, wspec, wspec, bspec, bspec],
        out_specs=nspec,
        out_shape=jax.ShapeDtypeStruct((_N, _H), jnp.float32),
    )(sp, cp, xn, x, wn2x, wn2a, Wnm, r2(bn2), r2(bnm))

    return (x_out, eo)


# speedup vs baseline: 1.1296x; 1.1296x over previous
import functools

import jax
import jax.numpy as jnp
from jax import lax
from jax.experimental import pallas as pl
from jax.experimental.pallas import tpu as pltpu
from jax.experimental.pallas import tpu_sc as plsc

_N = 10000
_E = 320000
_H = 128
_NC = 2
_NS = 16
_NW = _NC * _NS
_BX = 64
_BE = 1600
_BN = 2000


def _node_prep_body(x_ref, gn_ref, bn_ref, wes_ref, wed_ref, wn1s_ref,
                    xn_ref, p1_ref, p2_ref, p3_ref):
    x = x_ref[...]
    mu = jnp.mean(x, axis=1, keepdims=True)
    xc = x - mu
    var = jnp.mean(xc * xc, axis=1, keepdims=True)
    xn = xc * lax.rsqrt(var + 1e-5) * gn_ref[...] + bn_ref[...]
    xn_ref[...] = xn
    p1_ref[...] = jnp.dot(xn, wes_ref[...], preferred_element_type=jnp.float32)
    p2_ref[...] = jnp.dot(xn, wed_ref[...], preferred_element_type=jnp.float32)
    p3_ref[...] = jnp.dot(xn, wn1s_ref[...], preferred_element_type=jnp.float32)


def _edge_mlp_body(ea_ref, gs_ref, ge_ref, bee_ref, be_ref, bn1_ref,
                   bem_ref, wee_ref, wn1e_ref, wem_ref, eo_ref, c_ref):
    gs = gs_ref[...]
    ea = ea_ref[0]
    mu = jnp.mean(ea, axis=1, keepdims=True)
    xc = ea - mu
    var = jnp.mean(xc * xc, axis=1, keepdims=True)
    en = xc * lax.rsqrt(var + 1e-5) * ge_ref[...] + bee_ref[...]
    e = jnp.maximum(
        gs + jnp.dot(en, wee_ref[...], preferred_element_type=jnp.float32)
        + be_ref[...], 0.0)
    c_ref[...] = jnp.dot(e, wn1e_ref[...], preferred_element_type=jnp.float32) + bn1_ref[...]
    se = e / (1.0 + jnp.exp(-e))
    eo_ref[...] = ea + jnp.dot(se, wem_ref[...], preferred_element_type=jnp.float32) + bem_ref[...]


def _node_update_body(sp_ref, cp_ref, xn_ref, x_ref, wx_ref, wa_ref, wm_ref,
                      bn2_ref, bnm_ref, xo_ref):
    sums = jnp.sum(sp_ref[...], axis=0)
    cnt = jnp.sum(cp_ref[...], axis=0)[:, 0:1]
    agg = sums / jnp.maximum(cnt, 1.0)
    h = jnp.maximum(
        jnp.dot(xn_ref[...], wx_ref[...], preferred_element_type=jnp.float32)
        + jnp.dot(agg, wa_ref[...], preferred_element_type=jnp.float32)
        + bn2_ref[...], 0.0)
    sh = h / (1.0 + jnp.exp(-h))
    xo_ref[...] = x_ref[...] + jnp.dot(sh, wm_ref[...], preferred_element_type=jnp.float32) + bnm_ref[...]


def _tile_chunks(n, bx):
    assert n % _NS == 0
    rpt = n // _NS
    rptc = (rpt + 7) // 8 * 8
    assert rptc <= rpt + ((_NS - 1) * rpt) % 8
    chunks = []
    left = rptc
    while left > 0:
        chunks.append(min(bx, left))
        left -= chunks[-1]
    return rpt, chunks


def _build_gather(n, e, h, bx, interpret=False):
    nblk = e // bx
    rpt, chunks = _tile_chunks(n, bx)
    mesh = plsc.VectorSubcoreMesh(core_axis_name="c", subcore_axis_name="s")

    @functools.partial(
        pl.kernel,
        out_type=(jax.ShapeDtypeStruct((e, h), jnp.float32),
                  jax.ShapeDtypeStruct((_NC * n, h), jnp.float32)),
        mesh=mesh,
        scratch_types=[
            pltpu.VMEM((bx,), jnp.int32),
            pltpu.VMEM((bx,), jnp.int32),
            pltpu.VMEM((bx,), jnp.int32),
            pltpu.VMEM((bx,), jnp.int32),
            pltpu.VMEM((bx, h), jnp.float32),
            pltpu.VMEM((bx, h), jnp.float32),
            pltpu.VMEM((bx, h), jnp.float32),
            pltpu.VMEM((bx, h), jnp.float32),
            pltpu.VMEM((bx, h), jnp.float32),
            pltpu.VMEM_SHARED((n, h), jnp.float32),
            pltpu.SemaphoreType.DMA,
            pltpu.SemaphoreType.DMA,
            pltpu.SemaphoreType.DMA,
            pltpu.SemaphoreType.DMA,
            pltpu.SemaphoreType.DMA,
            pltpu.SemaphoreType.DMA,
        ],
        interpret=interpret,
    )
    def gather_add(p1_hbm, p2_hbm, row_hbm, col_hbm, out_hbm, cnt_hbm,
                   idxr0, idxr1, idxc0, idxc1, bufa0, bufa1, bufb0, bufb1,
                   onesb, cntacc,
                   semi0, semi1, semg0, semg1, semw0, semw1):
        idxr = (idxr0, idxr1)
        idxc = (idxc0, idxc1)
        bufa = (bufa0, bufa1)
        bufb = (bufb0, bufb1)
        semi = (semi0, semi1)
        semg = (semg0, semg1)
        semw = (semw0, semw1)
        cid = lax.axis_index("c")
        sid = lax.axis_index("s")
        wid = sid * _NC + cid

        def zrow(i, c2):
            for jj in range(h // 16):
                onesb[i, pl.ds(jj * 16, 16)] = jnp.zeros((16,), jnp.float32)
            return c2

        lax.fori_loop(0, bx, zrow, 0)
        r0 = sid * rpt - lax.rem(sid * rpt, 8)
        ofs = 0
        for step in chunks:
            pltpu.sync_copy(onesb.at[pl.ds(0, step)], cntacc.at[pl.ds(r0 + ofs, step)])
            ofs += step

        def onerow(i, c2):
            for jj in range(h // 16):
                onesb[i, pl.ds(jj * 16, 16)] = jnp.full((16,), 1.0, jnp.float32)
            return c2

        lax.fori_loop(0, bx, onerow, 0)
        plsc.subcore_barrier()

        nblk_w = (nblk - 1 - wid) // _NW + 1

        def off_of(k):
            return pl.multiple_of((wid + k * _NW) * bx, bx)

        def issue_idx(k, s):
            pltpu.async_copy(row_hbm.at[pl.ds(off_of(k), bx)], idxr[s], semi[s])
            pltpu.async_copy(col_hbm.at[pl.ds(off_of(k), bx)], idxc[s], semi[s])

        def wait_idx(s):
            pltpu.make_async_copy(row_hbm.at[pl.ds(0, bx)], idxr[s], semi[s]).wait()
            pltpu.make_async_copy(col_hbm.at[pl.ds(0, bx)], idxc[s], semi[s]).wait()

        def issue_gather(s):
            pltpu.async_copy(p1_hbm.at[idxr[s]], bufa[s], semg[s])
            pltpu.async_copy(p2_hbm.at[idxc[s]], bufb[s], semg[s])

        def wait_gather(s):
            pltpu.make_async_copy(p1_hbm.at[idxr[s]], bufa[s], semg[s]).wait()
            pltpu.make_async_copy(p2_hbm.at[idxc[s]], bufb[s], semg[s]).wait()

        def issue_write(k, s):
            pltpu.async_copy(bufa[s], out_hbm.at[pl.ds(off_of(k), bx)], semw[s])

        def wait_write(s):
            pltpu.make_async_copy(bufa[s], out_hbm.at[pl.ds(0, bx)], semw[s]).wait()

        @pl.when(nblk_w > 0)
        def _():
            issue_idx(0, 0)

        @pl.when(nblk_w > 1)
        def _():
            issue_idx(1, 1)

        @pl.when(nblk_w > 0)
        def _():
            wait_idx(0)
            issue_gather(0)

        def outer(j, carry):
            for b in range(2):
                k = 2 * j + b
                p = b
                q = 1 - b
                kn = k + 1

                @pl.when(kn < nblk_w)
                def _():
                    wait_idx(q)

                @pl.when((k >= 1) & (k < nblk_w))
                def _():
                    wait_write(q)

                @pl.when(kn < nblk_w)
                def _():
                    issue_gather(q)

                @pl.when(k < nblk_w)
                def _():
                    wait_gather(p)

                    def addrow(i, c2):
                        for jj in range(h // 16):
                            sl = pl.ds(jj * 16, 16)
                            bufa[p][i, sl] = bufa[p][i, sl] + bufb[p][i, sl]
                        return c2

                    lax.fori_loop(0, bx, addrow, 0)
                    pltpu.sync_copy(onesb, cntacc.at[idxc[p]], add=True)

                @pl.when(k + 2 < nblk_w)
                def _():
                    issue_idx(k + 2, p)

                @pl.when(k < nblk_w)
                def _():
                    issue_write(k, p)
            return carry

        lax.fori_loop(0, (nblk_w + 1) // 2, outer, 0)

        last = nblk_w - 1

        @pl.when((last >= 0) & (lax.rem(last, 2) == 0))
        def _():
            wait_write(0)

        @pl.when((last >= 0) & (lax.rem(last, 2) == 1))
        def _():
            wait_write(1)

        plsc.subcore_barrier()

        ofs = 0
        for step in chunks:
            woff = pl.multiple_of(cid * n + r0 + ofs, 8)
            pltpu.sync_copy(cntacc.at[pl.ds(r0 + ofs, step)], onesb.at[pl.ds(0, step)])
            pltpu.sync_copy(onesb.at[pl.ds(0, step)], cnt_hbm.at[pl.ds(woff, step)])
            ofs += step

    return gather_add


def _build_scatter(n, e, h, bx, interpret=False):
    nblk = e // bx
    rpt, chunks = _tile_chunks(n, bx)
    mesh = plsc.VectorSubcoreMesh(core_axis_name="c", subcore_axis_name="s")

    @functools.partial(
        pl.kernel,
        out_type=jax.ShapeDtypeStruct((_NC * n, h), jnp.float32),
        mesh=mesh,
        scratch_types=[
            pltpu.VMEM((bx,), jnp.int32),
            pltpu.VMEM((bx,), jnp.int32),
            pltpu.VMEM((bx,), jnp.int32),
            pltpu.VMEM((bx,), jnp.int32),
            pltpu.VMEM((bx, h), jnp.float32),
            pltpu.VMEM((bx, h), jnp.float32),
            pltpu.VMEM((bx, h), jnp.float32),
            pltpu.VMEM((bx, h), jnp.float32),
            pltpu.VMEM_SHARED((n, h), jnp.float32),
            pltpu.SemaphoreType.DMA,
            pltpu.SemaphoreType.DMA,
            pltpu.SemaphoreType.DMA,
            pltpu.SemaphoreType.DMA,
            pltpu.SemaphoreType.DMA,
            pltpu.SemaphoreType.DMA,
        ],
        interpret=interpret,
    )
    def scatter_mean(p3_hbm, c_hbm, row_hbm, col_hbm, sums_hbm,
                     idxr0, idxr1, idxc0, idxc1, cbuf0, cbuf1, gbuf0, gbuf1,
                     acc, semi0, semi1, semc0, semc1, semg0, semg1):
        idxr = (idxr0, idxr1)
        idxc = (idxc0, idxc1)
        cbuf = (cbuf0, cbuf1)
        gbuf = (gbuf0, gbuf1)
        semi = (semi0, semi1)
        semc = (semc0, semc1)
        semg = (semg0, semg1)
        cid = lax.axis_index("c")
        sid = lax.axis_index("s")
        wid = sid * _NC + cid

        def zrow(i, c2):
            for j in range(h // 16):
                cbuf0[i, pl.ds(j * 16, 16)] = jnp.zeros((16,), jnp.float32)
            return c2

        lax.fori_loop(0, bx, zrow, 0)

        r0 = sid * rpt - lax.rem(sid * rpt, 8)
        ofs = 0
        for step in chunks:
            pltpu.sync_copy(cbuf0.at[pl.ds(0, step)], acc.at[pl.ds(r0 + ofs, step)])
            ofs += step
        plsc.subcore_barrier()

        nblk_w = (nblk - 1 - wid) // _NW + 1

        def off_of(k):
            return pl.multiple_of((wid + k * _NW) * bx, bx)

        def issue_idx(k, s):
            pltpu.async_copy(row_hbm.at[pl.ds(off_of(k), bx)], idxr[s], semi[s])
            pltpu.async_copy(col_hbm.at[pl.ds(off_of(k), bx)], idxc[s], semi[s])

        def wait_idx(s):
            pltpu.make_async_copy(row_hbm.at[pl.ds(0, bx)], idxr[s], semi[s]).wait()
            pltpu.make_async_copy(col_hbm.at[pl.ds(0, bx)], idxc[s], semi[s]).wait()

        def issue_data(k, s):
            pltpu.async_copy(c_hbm.at[pl.ds(off_of(k), bx)], cbuf[s], semc[s])
            pltpu.async_copy(p3_hbm.at[idxr[s]], gbuf[s], semg[s])

        def wait_data(s):
            pltpu.make_async_copy(c_hbm.at[pl.ds(0, bx)], cbuf[s], semc[s]).wait()
            pltpu.make_async_copy(p3_hbm.at[idxr[s]], gbuf[s], semg[s]).wait()

        @pl.when(nblk_w > 0)
        def _():
            issue_idx(0, 0)

        @pl.when(nblk_w > 1)
        def _():
            issue_idx(1, 1)

        @pl.when(nblk_w > 0)
        def _():
            wait_idx(0)
            issue_data(0, 0)

        def outer(j, carry):
            for b in range(2):
                k = 2 * j + b
                p = b
                q = 1 - b
                kn = k + 1

                @pl.when(kn < nblk_w)
                def _():
                    wait_idx(q)
                    issue_data(kn, q)

                @pl.when(k < nblk_w)
                def _():
                    wait_data(p)

                    def mrow(i, c2):
                        for jj in range(h // 16):
                            sl = pl.ds(jj * 16, 16)
                            cbuf[p][i, sl] = jnp.maximum(cbuf[p][i, sl] + gbuf[p][i, sl], 0.0)
                        return c2

                    lax.fori_loop(0, bx, mrow, 0)
                    pltpu.sync_copy(cbuf[p], acc.at[idxc[p]], add=True)

                @pl.when(k + 2 < nblk_w)
                def _():
                    issue_idx(k + 2, p)
            return carry

        lax.fori_loop(0, (nblk_w + 1) // 2, outer, 0)
        plsc.subcore_barrier()

        ofs = 0
        for step in chunks:
            woff = pl.multiple_of(cid * n + r0 + ofs, 8)
            pltpu.sync_copy(acc.at[pl.ds(r0 + ofs, step)], cbuf0.at[pl.ds(0, step)])
            pltpu.sync_copy(cbuf0.at[pl.ds(0, step)], sums_hbm.at[pl.ds(woff, step)])
            ofs += step

    return scatter_mean


_NSPLIT = 2


@functools.cache
def _gather_sc_cached():
    return _build_gather(_N, _E // _NSPLIT, _H, _BX)


@functools.cache
def _scatter_sc_cached():
    return _build_scatter(_N, _E // _NSPLIT, _H, 64)


def _gather_sc(p1, p2, row, col):
    return _gather_sc_cached()(p1, p2, row, col)


def _scatter_sc(p3, c, row, col):
    return _scatter_sc_cached()(p3, c, row, col)


def kernel(x, edge_index, edge_attr, u, batch, g_n, b_n, g_e, b_e, We, be,
           Wn1, bn1, Wn2, bn2, Wnm, bnm, Wem, bem):
    del u, batch
    row = edge_index[0]
    col = edge_index[1]
    wes, wed, wee = We[:_H], We[_H:2 * _H], We[2 * _H:]
    wn1s, wn1e = Wn1[:_H], Wn1[_H:]
    wn2x, wn2a = Wn2[:_H], Wn2[_H:]
    r2 = lambda v: v.reshape(1, _H)

    xn, p1, p2, p3 = pl.pallas_call(
        _node_prep_body,
        out_shape=[jax.ShapeDtypeStruct((_N, _H), jnp.float32)] * 4,
    )(x, r2(g_n), r2(b_n), wes, wed, wn1s)

    e2 = _E // _NSPLIT
    ea3 = edge_attr.reshape(_NSPLIT, e2, _H)
    wspec = pl.BlockSpec((_H, _H), lambda i: (0, 0))
    bspec = pl.BlockSpec((1, _H), lambda i: (0, 0))
    espec = pl.BlockSpec((_BE, _H), lambda i: (i, 0))

    gs_c = [_gather_sc(p1, p2, row[t * e2:(t + 1) * e2],
                       col[t * e2:(t + 1) * e2]) for t in range(_NSPLIT)]

    eo_c = []
    c_c = []
    for t in range(_NSPLIT):
        eo_t, c_t = pl.pallas_call(
            _edge_mlp_body,
            grid=(e2 // _BE,),
            in_specs=[pl.BlockSpec((1, _BE, _H), lambda i, t=t: (t, i, 0)),
                      espec, bspec, bspec, bspec, bspec, bspec,
                      wspec, wspec, wspec],
            out_specs=[espec, espec],
            out_shape=[jax.ShapeDtypeStruct((e2, _H), jnp.float32)] * 2,
        )(ea3, gs_c[t][0], r2(g_e), r2(b_e), r2(be), r2(bn1), r2(bem),
          wee, wn1e, Wem)
        eo_c.append(eo_t)
        c_c.append(c_t)
    eo = jnp.concatenate(eo_c, axis=0)

    sums_c = [_scatter_sc(p3, c_c[t], row[t * e2:(t + 1) * e2],
                          col[t * e2:(t + 1) * e2]) for t in range(_NSPLIT)]

    sp = jnp.concatenate(sums_c, axis=0).reshape(_NSPLIT * _NC, _N, _H)
    cp = jnp.concatenate([g[1] for g in gs_c], axis=0).reshape(
        _NSPLIT * _NC, _N, _H)

    nparts = _NSPLIT * _NC
    nspec = pl.BlockSpec((_BN, _H), lambda i: (i, 0))
    x_out = pl.pallas_call(
        _node_update_body,
        grid=(_N // _BN,),
        in_specs=[pl.BlockSpec((nparts, _BN, _H), lambda i: (0, i, 0)),
                  pl.BlockSpec((nparts, _BN, _H), lambda i: (0, i, 0)),
                  nspec, nspec, wspec, wspec, wspec, bspec, bspec],
        out_specs=nspec,
        out_shape=jax.ShapeDtypeStruct((_N, _H), jnp.float32),
    )(sp, cp, xn, x, wn2x, wn2a, Wnm, r2(bn2), r2(bnm))

    return (x_out, eo)
